# baseline (speedup 1.0000x reference)
import functools

import jax
import jax.numpy as jnp
from jax import lax
from jax.experimental import pallas as pl
from jax.experimental.pallas import tpu as pltpu
from jax.experimental.pallas import tpu_sc as plsc

B, H, W, C = 8, 32, 32, 768
WS = 8
NH = 12
DH = C // NH
E = 8
HID = 3072
N_TOK = B * H * W
WB = 4
TB = WB * WS * WS
N_BLK = N_TOK // TB
TM = 256
MAX_T = N_TOK // TM + E
NEG = -1e9

_bf = jnp.bfloat16
_f32 = jnp.float32


def _dot(a, b, dims, prec=None):
    return lax.dot_general(a, b, (dims, ((), ())), precision=prec,
                           preferred_element_type=_f32)


def _bdot(a, b, dims):
    return _dot(a.astype(_bf), b.astype(_bf), dims)


def _bias_kernel(onehot_ref, table_ref, out_ref):
    out_ref[:, :] = _dot(onehot_ref[:, :], table_ref[:, :], ((1,), (0,)),
                         prec=lax.Precision.HIGHEST)


def _rpb_bias(rpb_table, rel_pos_index):
    onehot = (rel_pos_index[:, None]
              == jnp.arange((2 * WS - 1) ** 2, dtype=rel_pos_index.dtype))
    onehot = onehot.astype(_f32)
    out = pl.pallas_call(
        _bias_kernel,
        out_shape=jax.ShapeDtypeStruct((WS * WS * WS * WS, NH), _f32),
    )(onehot, rpb_table)
    bias = out.reshape(WS * WS, WS * WS, NH).transpose(2, 0, 1)
    p = jnp.arange(TB)
    r, w = p // W, p % W
    wj = w // WS
    a = r * WS + w % WS
    mask = wj[:, None] == wj[None, :]
    big = jnp.where(mask[None], bias[:, a[:, None], a[None, :]], NEG)
    return big.astype(_f32)


def _attn_kernel(xw_ref, qkvw_ref, qkvb_ref, projw_ref, projb_ref,
                 n1g_ref, n1b_ref, n2g_ref, n2b_ref, gw_ref, gb_ref,
                 bias_ref, xmid_ref, h2_ref, idx_ref):
    x = xw_ref[:, :]
    m = jnp.mean(x, axis=-1, keepdims=True)
    v = jnp.mean((x - m) ** 2, axis=-1, keepdims=True)
    h = (x - m) / jnp.sqrt(v + 1e-5) * n1g_ref[:, :] + n1b_ref[:, :]

    qkv = _bdot(h, qkvw_ref[:, :], ((1,), (1,))) + qkvb_ref[:, :]

    outs = []
    for hd in range(NH):
        q = qkv[:, hd * DH:(hd + 1) * DH] * (DH ** -0.5)
        k = qkv[:, C + hd * DH:C + (hd + 1) * DH]
        vv = qkv[:, 2 * C + hd * DH:2 * C + (hd + 1) * DH]
        s = _bdot(q, k, ((1,), (1,))) + bias_ref[hd]
        mx = jnp.max(s, axis=-1, keepdims=True)
        e = jnp.exp(s - mx)
        p = e / jnp.sum(e, axis=-1, keepdims=True)
        outs.append(_bdot(p, vv, ((1,), (0,))))
    o = jnp.concatenate(outs, axis=1)

    po = _bdot(o, projw_ref[:, :], ((1,), (1,))) + projb_ref[:, :]
    xm = x + po
    xmid_ref[:, :] = xm

    m2 = jnp.mean(xm, axis=-1, keepdims=True)
    v2 = jnp.mean((xm - m2) ** 2, axis=-1, keepdims=True)
    h2 = (xm - m2) / jnp.sqrt(v2 + 1e-5) * n2g_ref[:, :] + n2b_ref[:, :]
    h2_ref[:, :] = h2.astype(_bf)

    g = _bdot(h2, gw_ref[:, :], ((1,), (1,))) + gb_ref[:, :]
    mxg = jnp.max(g, axis=1, keepdims=True)
    iota = lax.broadcasted_iota(jnp.int32, (TB, E), 1)
    idx = jnp.min(jnp.where(g == mxg, iota, E), axis=1)
    idx_ref[0, 0, :] = idx.astype(jnp.int32)


def _attention(xw, qkv_w, qkv_b, proj_w, proj_b, n1g, n1b, n2g, n2b,
               gate_w, gate_b, big_bias):
    row = lambda a: a.reshape(1, -1)
    wargs = [qkv_w.astype(_bf), row(qkv_b), proj_w.astype(_bf), row(proj_b),
             row(n1g), row(n1b), row(n2g), row(n2b),
             gate_w.astype(_bf), row(gate_b), big_bias]
    full = lambda a: pl.BlockSpec(a.shape, lambda i, nd=a.ndim: (0,) * nd)
    return pl.pallas_call(
        _attn_kernel,
        grid=(N_BLK,),
        in_specs=[pl.BlockSpec((TB, C), lambda i: (i, 0))] +
                 [full(a) for a in wargs],
        out_specs=[
            pl.BlockSpec((TB, C), lambda i: (i, 0)),
            pl.BlockSpec((TB, C), lambda i: (i, 0)),
            pl.BlockSpec((1, 1, TB), lambda i: (i, 0, 0)),
        ],
        out_shape=[
            jax.ShapeDtypeStruct((N_TOK, C), _f32),
            jax.ShapeDtypeStruct((N_TOK, C), _bf),
            jax.ShapeDtypeStruct((N_BLK, 1, TB), jnp.int32),
        ],
    )(xw, *wargs)


def _routing(idx):
    eye = jnp.arange(E, dtype=jnp.int32)
    onehot = (idx[:, None] == eye[None, :]).astype(jnp.int32)
    counts = jnp.sum(onehot, axis=0)
    rank = jnp.cumsum(onehot, axis=0) - onehot
    rank_i = jnp.sum(rank * onehot, axis=1)
    tiles_e = (counts + TM - 1) // TM
    t_end = jnp.cumsum(tiles_e)
    t_off = t_end - tiles_e
    p_off = t_off * TM
    slot = (p_off[idx] + rank_i).astype(jnp.int32)

    gather_idx = jnp.zeros((MAX_T * TM,), jnp.int32).at[slot].set(
        jnp.arange(N_TOK, dtype=jnp.int32), mode='drop')

    tt = jnp.arange(MAX_T, dtype=jnp.int32)
    e_t = jnp.minimum(jnp.sum((tt[:, None] >= t_end[None, :]).astype(jnp.int32),
                              axis=1), E - 1).astype(jnp.int32)
    return e_t, gather_idx, slot


_NW = 32


def _sc_gather(table, idx, chunk):
    if table.dtype == _bf:
        t32 = lax.bitcast_convert_type(
            table.reshape(table.shape[0], table.shape[1] // 2, 2), jnp.int32)
        out32 = _sc_gather(t32, idx, chunk)
        return lax.bitcast_convert_type(out32, _bf).reshape(
            idx.shape[0], table.shape[1])
    rows = idx.shape[0]
    d = table.shape[1]
    bpw = rows // _NW
    nst = bpw // chunk
    mesh = plsc.VectorSubcoreMesh(core_axis_name="c", subcore_axis_name="s")

    @functools.partial(
        pl.kernel, mesh=mesh,
        out_type=jax.ShapeDtypeStruct((rows, d), table.dtype),
        scratch_types=[
            pltpu.VMEM((chunk,), jnp.int32),
            pltpu.VMEM((chunk, d), table.dtype),
            pltpu.SemaphoreType.DMA,
        ],
    )
    def k(table_hbm, idx_hbm, out_hbm, idx_v, rows_v, sem):
        wid = lax.axis_index("s") * 2 + lax.axis_index("c")

        def body(j, c):
            base = wid * bpw + j * chunk
            pltpu.sync_copy(idx_hbm.at[pl.ds(base, chunk)], idx_v)
            pltpu.async_copy(table_hbm.at[idx_v], rows_v, sem).wait()
            pltpu.sync_copy(rows_v, out_hbm.at[pl.ds(base, chunk)])
            return c
        lax.fori_loop(0, nst, body, 0)

    return k(table, idx)


def _moe_kernel(te_ref, xg_ref, w1_ref, b1_ref, w2_ref, b2_ref, out_ref):
    x = xg_ref[:, :]
    hmid = _dot(x, w1_ref[0], ((1,), (1,))) + b1_ref[0]
    a = hmid * 0.5 * (1.0 + lax.erf(hmid * (2.0 ** -0.5)))
    o = _dot(a.astype(_bf), w2_ref[0], ((1,), (1,))) + b2_ref[0]
    out_ref[:, :] = o.astype(_bf)


def _moe(xg, fc1_w, fc1_b, fc2_w, fc2_b, te):
    grid_spec = pltpu.PrefetchScalarGridSpec(
        num_scalar_prefetch=1,
        grid=(MAX_T,),
        in_specs=[
            pl.BlockSpec((TM, C), lambda t, te: (t, 0)),
            pl.BlockSpec((1, HID, C), lambda t, te: (te[t], 0, 0)),
            pl.BlockSpec((1, 1, HID), lambda t, te: (te[t], 0, 0)),
            pl.BlockSpec((1, C, HID), lambda t, te: (te[t], 0, 0)),
            pl.BlockSpec((1, 1, C), lambda t, te: (te[t], 0, 0)),
        ],
        out_specs=pl.BlockSpec((TM, C), lambda t, te: (t, 0)),
    )
    return pl.pallas_call(
        _moe_kernel,
        grid_spec=grid_spec,
        out_shape=jax.ShapeDtypeStruct((MAX_T * TM, C), _bf),
    )(te, xg, fc1_w.astype(_bf), fc1_b.reshape(E, 1, HID),
      fc2_w.astype(_bf), fc2_b.reshape(E, 1, C))


@functools.partial(jax.jit, static_argnums=())
def kernel(x, norm1_g, norm1_b, qkv_w, qkv_b, rpb_table, proj_w, proj_b,
           norm2_g, norm2_b, gate_w, gate_b, fc1_w, fc1_b, fc2_w, fc2_b,
           rel_pos_index):
    big_bias = _rpb_bias(rpb_table, rel_pos_index)
    xw = x.reshape(N_TOK, C)
    xmid, h2, idx3 = _attention(
        xw, qkv_w, qkv_b, proj_w, proj_b, norm1_g, norm1_b,
        norm2_g, norm2_b, gate_w, gate_b, big_bias)
    idx = idx3.reshape(N_TOK)
    te, gather_idx, slot = _routing(idx)
    xg = _sc_gather(h2, gather_idx, 160)
    moe_sorted = _moe(xg, fc1_w, fc1_b, fc2_w, fc2_b, te)
    moe = _sc_gather(moe_sorted, slot, 256)
    y = (xmid + moe.astype(_f32)).reshape(B, H * W, C)
    return (y, jnp.float32(0.0))


# speedup vs baseline: 3.5290x; 3.5290x over previous
import functools

import jax
import jax.numpy as jnp
from jax import lax
from jax.experimental import pallas as pl
from jax.experimental.pallas import tpu as pltpu
from jax.experimental.pallas import tpu_sc as plsc

B, H, W, C = 8, 32, 32, 768
WS = 8
NH = 12
DH = C // NH
E = 8
HID = 3072
N_TOK = B * H * W
WB = 4
TB = WB * WS * WS
N_BLK = N_TOK // TB
TM = 256
MAX_T = N_TOK // TM + E
NEG = -1e9

_bf = jnp.bfloat16
_f32 = jnp.float32


def _dot(a, b, dims, prec=None):
    return lax.dot_general(a, b, (dims, ((), ())), precision=prec,
                           preferred_element_type=_f32)


def _bdot(a, b, dims):
    return _dot(a.astype(_bf), b.astype(_bf), dims)


def _bias_kernel(onehot_ref, table_ref, out_ref):
    out_ref[:, :] = _dot(onehot_ref[:, :], table_ref[:, :], ((1,), (0,)),
                         prec=lax.Precision.HIGHEST)


def _rpb_bias(rpb_table, rel_pos_index):
    onehot = (rel_pos_index[:, None]
              == jnp.arange((2 * WS - 1) ** 2, dtype=rel_pos_index.dtype))
    onehot = onehot.astype(_f32)
    out = pl.pallas_call(
        _bias_kernel,
        out_shape=jax.ShapeDtypeStruct((WS * WS * WS * WS, NH), _f32),
    )(onehot, rpb_table)
    bias = out.reshape(WS * WS, WS * WS, NH).transpose(2, 0, 1)
    p = jnp.arange(TB)
    r, w = p // W, p % W
    wj = w // WS
    a = r * WS + w % WS
    mask = wj[:, None] == wj[None, :]
    perm = (a[:, None] == jnp.arange(WS * WS)[None, :]).astype(_f32)
    big_p = jnp.einsum('pa,hab,qb->hpq', perm, bias, perm,
                       precision=lax.Precision.HIGHEST)
    big = jnp.where(mask[None], big_p, NEG)
    return big.astype(_f32)


def _attn_kernel(xw_ref, qkvw_ref, qkvb_ref, projw_ref, projb_ref,
                 n1g_ref, n1b_ref, n2g_ref, n2b_ref, gw_ref, gb_ref,
                 bias_ref, xmid_ref, h2_ref, idx_ref):
    x = xw_ref[:, :]
    m = jnp.mean(x, axis=-1, keepdims=True)
    v = jnp.mean((x - m) ** 2, axis=-1, keepdims=True)
    h = (x - m) / jnp.sqrt(v + 1e-5) * n1g_ref[:, :] + n1b_ref[:, :]

    qkv = _bdot(h, qkvw_ref[:, :], ((1,), (1,))) + qkvb_ref[:, :]

    outs = []
    for hd in range(NH):
        q = qkv[:, hd * DH:(hd + 1) * DH] * (DH ** -0.5)
        k = qkv[:, C + hd * DH:C + (hd + 1) * DH]
        vv = qkv[:, 2 * C + hd * DH:2 * C + (hd + 1) * DH]
        s = _bdot(q, k, ((1,), (1,))) + bias_ref[hd]
        mx = jnp.max(s, axis=-1, keepdims=True)
        e = jnp.exp(s - mx)
        p = e / jnp.sum(e, axis=-1, keepdims=True)
        outs.append(_bdot(p, vv, ((1,), (0,))))
    o = jnp.concatenate(outs, axis=1)

    po = _bdot(o, projw_ref[:, :], ((1,), (1,))) + projb_ref[:, :]
    xm = x + po
    xmid_ref[:, :] = xm

    m2 = jnp.mean(xm, axis=-1, keepdims=True)
    v2 = jnp.mean((xm - m2) ** 2, axis=-1, keepdims=True)
    h2 = (xm - m2) / jnp.sqrt(v2 + 1e-5) * n2g_ref[:, :] + n2b_ref[:, :]
    h2_ref[:, :] = h2

    g = _bdot(h2, gw_ref[:, :], ((1,), (1,))) + gb_ref[:, :]
    mxg = jnp.max(g, axis=1, keepdims=True)
    iota = lax.broadcasted_iota(jnp.int32, (TB, E), 1)
    idx = jnp.min(jnp.where(g == mxg, iota, E), axis=1)
    idx_ref[0, 0, :] = idx.astype(jnp.int32)


def _attention(xw, qkv_w, qkv_b, proj_w, proj_b, n1g, n1b, n2g, n2b,
               gate_w, gate_b, big_bias):
    row = lambda a: a.reshape(1, -1)
    wargs = [qkv_w.astype(_bf), row(qkv_b), proj_w.astype(_bf), row(proj_b),
             row(n1g), row(n1b), row(n2g), row(n2b),
             gate_w.astype(_bf), row(gate_b), big_bias]
    full = lambda a: pl.BlockSpec(a.shape, lambda i, nd=a.ndim: (0,) * nd)
    return pl.pallas_call(
        _attn_kernel,
        grid=(N_BLK,),
        in_specs=[pl.BlockSpec((TB, C), lambda i: (i, 0))] +
                 [full(a) for a in wargs],
        out_specs=[
            pl.BlockSpec((TB, C), lambda i: (i, 0)),
            pl.BlockSpec((TB, C), lambda i: (i, 0)),
            pl.BlockSpec((1, 1, TB), lambda i: (i, 0, 0)),
        ],
        out_shape=[
            jax.ShapeDtypeStruct((N_TOK, C), _f32),
            jax.ShapeDtypeStruct((N_TOK, C), _f32),
            jax.ShapeDtypeStruct((N_BLK, 1, TB), jnp.int32),
        ],
    )(xw, *wargs)


def _routing(idx):
    eye = jnp.arange(E, dtype=jnp.int32)
    onehot = (idx[:, None] == eye[None, :]).astype(jnp.int32)
    counts = jnp.sum(onehot, axis=0)
    rank = jnp.cumsum(onehot, axis=0) - onehot
    rank_i = jnp.sum(rank * onehot, axis=1)
    tiles_e = (counts + TM - 1) // TM
    t_end = jnp.cumsum(tiles_e)
    t_off = t_end - tiles_e
    p_off = t_off * TM
    slot = (p_off[idx] + rank_i).astype(jnp.int32)

    gather_idx = jnp.zeros((MAX_T * TM,), jnp.int32).at[slot].set(
        jnp.arange(N_TOK, dtype=jnp.int32), mode='drop')

    tt = jnp.arange(MAX_T, dtype=jnp.int32)
    e_t = jnp.minimum(jnp.sum((tt[:, None] >= t_end[None, :]).astype(jnp.int32),
                              axis=1), E - 1).astype(jnp.int32)
    return e_t, gather_idx, slot


_NW = 32


def _sc_gather(table, idx, chunk):
    rows = idx.shape[0]
    d = table.shape[1]
    bpw = rows // _NW
    nst = bpw // chunk
    mesh = plsc.VectorSubcoreMesh(core_axis_name="c", subcore_axis_name="s")

    @functools.partial(
        pl.kernel, mesh=mesh,
        out_type=jax.ShapeDtypeStruct((rows, d), table.dtype),
        scratch_types=[
            pltpu.VMEM((chunk,), jnp.int32),
            pltpu.VMEM((chunk, d), table.dtype),
            pltpu.SemaphoreType.DMA,
        ],
    )
    def k(table_hbm, idx_hbm, out_hbm, idx_v, rows_v, sem):
        wid = lax.axis_index("s") * 2 + lax.axis_index("c")

        def body(j, c):
            base = wid * bpw + j * chunk
            pltpu.sync_copy(idx_hbm.at[pl.ds(base, chunk)], idx_v)
            pltpu.async_copy(table_hbm.at[idx_v], rows_v, sem).wait()
            pltpu.sync_copy(rows_v, out_hbm.at[pl.ds(base, chunk)])
            return c
        lax.fori_loop(0, nst, body, 0)

    return k(table, idx)


def _moe_kernel(te_ref, xg_ref, w1_ref, b1_ref, w2_ref, b2_ref, out_ref):
    x = xg_ref[:, :]
    hmid = _bdot(x, w1_ref[0], ((1,), (1,))) + b1_ref[0]
    a = hmid * 0.5 * (1.0 + lax.erf(hmid * (2.0 ** -0.5)))
    out_ref[:, :] = _dot(a.astype(_bf), w2_ref[0], ((1,), (1,))) + b2_ref[0]


def _moe(xg, fc1_w, fc1_b, fc2_w, fc2_b, te):
    grid_spec = pltpu.PrefetchScalarGridSpec(
        num_scalar_prefetch=1,
        grid=(MAX_T,),
        in_specs=[
            pl.BlockSpec((TM, C), lambda t, te: (t, 0)),
            pl.BlockSpec((1, HID, C), lambda t, te: (te[t], 0, 0)),
            pl.BlockSpec((1, 1, HID), lambda t, te: (te[t], 0, 0)),
            pl.BlockSpec((1, C, HID), lambda t, te: (te[t], 0, 0)),
            pl.BlockSpec((1, 1, C), lambda t, te: (te[t], 0, 0)),
        ],
        out_specs=pl.BlockSpec((TM, C), lambda t, te: (t, 0)),
    )
    return pl.pallas_call(
        _moe_kernel,
        grid_spec=grid_spec,
        out_shape=jax.ShapeDtypeStruct((MAX_T * TM, C), _f32),
    )(te, xg, fc1_w.astype(_bf), fc1_b.reshape(E, 1, HID),
      fc2_w.astype(_bf), fc2_b.reshape(E, 1, C))


@functools.partial(jax.jit, static_argnums=())
def kernel(x, norm1_g, norm1_b, qkv_w, qkv_b, rpb_table, proj_w, proj_b,
           norm2_g, norm2_b, gate_w, gate_b, fc1_w, fc1_b, fc2_w, fc2_b,
           rel_pos_index):
    big_bias = _rpb_bias(rpb_table, rel_pos_index)
    xw = x.reshape(N_TOK, C)
    xmid, h2, idx3 = _attention(
        xw, qkv_w, qkv_b, proj_w, proj_b, norm1_g, norm1_b,
        norm2_g, norm2_b, gate_w, gate_b, big_bias)
    idx = idx3.reshape(N_TOK)
    te, gather_idx, slot = _routing(idx)
    xg = _sc_gather(h2, gather_idx, 80)
    moe_sorted = _moe(xg, fc1_w, fc1_b, fc2_w, fc2_b, te)
    moe = _sc_gather(moe_sorted, slot, 128)
    y = (xmid + moe).reshape(B, H * W, C)
    return (y, jnp.float32(0.0))


# speedup vs baseline: 4.6510x; 1.3179x over previous
import functools

import jax
import jax.numpy as jnp
from jax import lax
from jax.experimental import pallas as pl
from jax.experimental.pallas import tpu as pltpu
from jax.experimental.pallas import tpu_sc as plsc

B, H, W, C = 8, 32, 32, 768
WS = 8
NH = 12
DH = C // NH
E = 8
HID = 3072
N_TOK = B * H * W
WB = 4
TB = WB * WS * WS
N_BLK = N_TOK // TB
TM = 256
MAX_T = N_TOK // TM + E
NEG = -1e9

_bf = jnp.bfloat16
_f32 = jnp.float32


def _dot(a, b, dims, prec=None):
    return lax.dot_general(a, b, (dims, ((), ())), precision=prec,
                           preferred_element_type=_f32)


def _bdot(a, b, dims):
    return _dot(a.astype(_bf), b.astype(_bf), dims)


def _bias_kernel(onehot_ref, table_ref, out_ref):
    out_ref[:, :] = _dot(onehot_ref[:, :], table_ref[:, :], ((1,), (0,)),
                         prec=lax.Precision.HIGHEST)


def _rpb_bias(rpb_table, rel_pos_index):
    onehot = (rel_pos_index[:, None]
              == jnp.arange((2 * WS - 1) ** 2, dtype=rel_pos_index.dtype))
    onehot = onehot.astype(_f32)
    out = pl.pallas_call(
        _bias_kernel,
        out_shape=jax.ShapeDtypeStruct((WS * WS * WS * WS, NH), _f32),
    )(onehot, rpb_table)
    bias = out.reshape(WS * WS, WS * WS, NH).transpose(2, 0, 1)
    p = jnp.arange(TB)
    r, w = p // W, p % W
    wj = w // WS
    a = r * WS + w % WS
    mask = wj[:, None] == wj[None, :]
    perm = (a[:, None] == jnp.arange(WS * WS)[None, :]).astype(_f32)
    big_p = jnp.einsum('pa,hab,qb->hpq', perm, bias, perm,
                       precision=lax.Precision.HIGHEST)
    big = jnp.where(mask[None], big_p, NEG)
    return big.astype(_f32)


def _attn_kernel(xw_ref, qkvw_ref, qkvb_ref, projw_ref, projb_ref,
                 n1g_ref, n1b_ref, n2g_ref, n2b_ref, gw_ref, gb_ref,
                 bias_ref, xmid_ref, h2_ref, idx_ref):
    x = xw_ref[:, :]
    m = jnp.mean(x, axis=-1, keepdims=True)
    v = jnp.mean((x - m) ** 2, axis=-1, keepdims=True)
    h = (x - m) / jnp.sqrt(v + 1e-5) * n1g_ref[:, :] + n1b_ref[:, :]

    qkv = _bdot(h, qkvw_ref[:, :], ((1,), (1,))) + qkvb_ref[:, :]

    outs = []
    for hd in range(NH):
        q = qkv[:, hd * DH:(hd + 1) * DH] * (DH ** -0.5)
        k = qkv[:, C + hd * DH:C + (hd + 1) * DH]
        vv = qkv[:, 2 * C + hd * DH:2 * C + (hd + 1) * DH]
        s = _bdot(q, k, ((1,), (1,))) + bias_ref[hd]
        mx = jnp.max(s, axis=-1, keepdims=True)
        e = jnp.exp(s - mx)
        p = e / jnp.sum(e, axis=-1, keepdims=True)
        outs.append(_bdot(p, vv, ((1,), (0,))))
    o = jnp.concatenate(outs, axis=1)

    po = _bdot(o, projw_ref[:, :], ((1,), (1,))) + projb_ref[:, :]
    xm = x + po
    xmid_ref[:, :] = xm

    m2 = jnp.mean(xm, axis=-1, keepdims=True)
    v2 = jnp.mean((xm - m2) ** 2, axis=-1, keepdims=True)
    h2 = (xm - m2) / jnp.sqrt(v2 + 1e-5) * n2g_ref[:, :] + n2b_ref[:, :]
    h2_ref[:, :] = h2

    g = _bdot(h2, gw_ref[:, :], ((1,), (1,))) + gb_ref[:, :]
    mxg = jnp.max(g, axis=1, keepdims=True)
    iota = lax.broadcasted_iota(jnp.int32, (TB, E), 1)
    idx = jnp.min(jnp.where(g == mxg, iota, E), axis=1)
    idx_ref[0, 0, :] = idx.astype(jnp.int32)


def _attention(xw, qkv_w, qkv_b, proj_w, proj_b, n1g, n1b, n2g, n2b,
               gate_w, gate_b, big_bias):
    row = lambda a: a.reshape(1, -1)
    wargs = [qkv_w.astype(_bf), row(qkv_b), proj_w.astype(_bf), row(proj_b),
             row(n1g), row(n1b), row(n2g), row(n2b),
             gate_w.astype(_bf), row(gate_b), big_bias]
    full = lambda a: pl.BlockSpec(a.shape, lambda i, nd=a.ndim: (0,) * nd)
    return pl.pallas_call(
        _attn_kernel,
        grid=(N_BLK,),
        in_specs=[pl.BlockSpec((TB, C), lambda i: (i, 0))] +
                 [full(a) for a in wargs],
        out_specs=[
            pl.BlockSpec((TB, C), lambda i: (i, 0)),
            pl.BlockSpec((TB, C), lambda i: (i, 0)),
            pl.BlockSpec((1, 1, TB), lambda i: (i, 0, 0)),
        ],
        out_shape=[
            jax.ShapeDtypeStruct((N_TOK, C), _f32),
            jax.ShapeDtypeStruct((N_TOK, C), _f32),
            jax.ShapeDtypeStruct((N_BLK, 1, TB), jnp.int32),
        ],
    )(xw, *wargs)


def _routing(idx):
    eye = jnp.arange(E, dtype=jnp.int32)
    onehot = (idx[:, None] == eye[None, :]).astype(jnp.int32)
    counts = jnp.sum(onehot, axis=0)
    rank = jnp.cumsum(onehot, axis=0) - onehot
    rank_i = jnp.sum(rank * onehot, axis=1)
    tiles_e = (counts + TM - 1) // TM
    t_end = jnp.cumsum(tiles_e)
    t_off = t_end - tiles_e
    p_off = t_off * TM
    p_off_tok = jnp.sum(onehot * p_off[None, :], axis=1)
    slot = (p_off_tok + rank_i).astype(jnp.int32)

    tt = jnp.arange(MAX_T, dtype=jnp.int32)
    e_t = jnp.minimum(jnp.sum((tt[:, None] >= t_end[None, :]).astype(jnp.int32),
                              axis=1), E - 1).astype(jnp.int32)
    return e_t, slot


_NW = 32


def _sc_gather(table, idx, chunk):
    rows = idx.shape[0]
    d = table.shape[1]
    bpw = rows // _NW
    nst = bpw // chunk
    mesh = plsc.VectorSubcoreMesh(core_axis_name="c", subcore_axis_name="s")

    @functools.partial(
        pl.kernel, mesh=mesh,
        out_type=jax.ShapeDtypeStruct((rows, d), table.dtype),
        scratch_types=[
            pltpu.VMEM((chunk,), jnp.int32),
            pltpu.VMEM((chunk, d), table.dtype),
            pltpu.SemaphoreType.DMA,
        ],
    )
    def k(table_hbm, idx_hbm, out_hbm, idx_v, rows_v, sem):
        wid = lax.axis_index("s") * 2 + lax.axis_index("c")

        def body(j, c):
            base = wid * bpw + j * chunk
            pltpu.sync_copy(idx_hbm.at[pl.ds(base, chunk)], idx_v)
            pltpu.async_copy(table_hbm.at[idx_v], rows_v, sem).wait()
            pltpu.sync_copy(rows_v, out_hbm.at[pl.ds(base, chunk)])
            return c
        lax.fori_loop(0, nst, body, 0)

    return k(table, idx)


def _sc_scatter(rows_hbm, slot, out_rows, chunk):
    n, d = rows_hbm.shape
    bpw = n // _NW
    nst = bpw // chunk
    mesh = plsc.VectorSubcoreMesh(core_axis_name="c", subcore_axis_name="s")

    @functools.partial(
        pl.kernel, mesh=mesh,
        out_type=jax.ShapeDtypeStruct((out_rows, d), rows_hbm.dtype),
        scratch_types=[
            pltpu.VMEM((chunk,), jnp.int32),
            pltpu.VMEM((chunk, d), rows_hbm.dtype),
            pltpu.SemaphoreType.DMA,
        ],
    )
    def k(rows_h, slot_h, out_hbm, idx_v, rows_v, sem):
        wid = lax.axis_index("s") * 2 + lax.axis_index("c")

        def body(j, c):
            base = wid * bpw + j * chunk
            pltpu.sync_copy(slot_h.at[pl.ds(base, chunk)], idx_v)
            pltpu.sync_copy(rows_h.at[pl.ds(base, chunk)], rows_v)
            pltpu.async_copy(rows_v, out_hbm.at[idx_v], sem).wait()
            return c
        lax.fori_loop(0, nst, body, 0)

    return k(rows_hbm, slot)


def _moe_kernel(te_ref, xg_ref, w1_ref, b1_ref, w2_ref, b2_ref, out_ref):
    x = xg_ref[:, :]
    hmid = _bdot(x, w1_ref[0], ((1,), (1,))) + b1_ref[0]
    a = hmid * 0.5 * (1.0 + lax.erf(hmid * (2.0 ** -0.5)))
    out_ref[:, :] = _dot(a.astype(_bf), w2_ref[0], ((1,), (1,))) + b2_ref[0]


def _moe(xg, fc1_w, fc1_b, fc2_w, fc2_b, te):
    grid_spec = pltpu.PrefetchScalarGridSpec(
        num_scalar_prefetch=1,
        grid=(MAX_T,),
        in_specs=[
            pl.BlockSpec((TM, C), lambda t, te: (t, 0)),
            pl.BlockSpec((1, HID, C), lambda t, te: (te[t], 0, 0)),
            pl.BlockSpec((1, 1, HID), lambda t, te: (te[t], 0, 0)),
            pl.BlockSpec((1, C, HID), lambda t, te: (te[t], 0, 0)),
            pl.BlockSpec((1, 1, C), lambda t, te: (te[t], 0, 0)),
        ],
        out_specs=pl.BlockSpec((TM, C), lambda t, te: (t, 0)),
    )
    return pl.pallas_call(
        _moe_kernel,
        grid_spec=grid_spec,
        out_shape=jax.ShapeDtypeStruct((MAX_T * TM, C), _f32),
    )(te, xg, fc1_w.astype(_bf), fc1_b.reshape(E, 1, HID),
      fc2_w.astype(_bf), fc2_b.reshape(E, 1, C))


@functools.partial(jax.jit, static_argnums=())
def kernel(x, norm1_g, norm1_b, qkv_w, qkv_b, rpb_table, proj_w, proj_b,
           norm2_g, norm2_b, gate_w, gate_b, fc1_w, fc1_b, fc2_w, fc2_b,
           rel_pos_index):
    big_bias = _rpb_bias(rpb_table, rel_pos_index)
    xw = x.reshape(N_TOK, C)
    xmid, h2, idx3 = _attention(
        xw, qkv_w, qkv_b, proj_w, proj_b, norm1_g, norm1_b,
        norm2_g, norm2_b, gate_w, gate_b, big_bias)
    idx = idx3.reshape(N_TOK)
    te, slot = _routing(idx)
    xg = _sc_scatter(h2, slot, MAX_T * TM, 128)
    moe_sorted = _moe(xg, fc1_w, fc1_b, fc2_w, fc2_b, te)
    moe = _sc_gather(moe_sorted, slot, 128)
    y = (xmid + moe).reshape(B, H * W, C)
    return (y, jnp.float32(0.0))


# speedup vs baseline: 4.9841x; 1.0716x over previous
import functools

import jax
import jax.numpy as jnp
from jax import lax
from jax.experimental import pallas as pl
from jax.experimental.pallas import tpu as pltpu
from jax.experimental.pallas import tpu_sc as plsc

B, H, W, C = 8, 32, 32, 768
WS = 8
NH = 12
DH = C // NH
E = 8
HID = 3072
N_TOK = B * H * W
WB = 4
TB = WB * WS * WS
N_BLK = N_TOK // TB
TM = 256
MAX_T = N_TOK // TM + E
NEG = -1e9

_bf = jnp.bfloat16
_f32 = jnp.float32


def _dot(a, b, dims, prec=None):
    return lax.dot_general(a, b, (dims, ((), ())), precision=prec,
                           preferred_element_type=_f32)


def _bdot(a, b, dims):
    return _dot(a.astype(_bf), b.astype(_bf), dims)


def _bias_kernel(onehot_ref, table_ref, out_ref):
    out_ref[:, :] = _dot(onehot_ref[:, :], table_ref[:, :], ((1,), (0,)),
                         prec=lax.Precision.HIGHEST)


def _rpb_bias(rpb_table, rel_pos_index):
    onehot = (rel_pos_index[:, None]
              == jnp.arange((2 * WS - 1) ** 2, dtype=rel_pos_index.dtype))
    onehot = onehot.astype(_f32)
    out = pl.pallas_call(
        _bias_kernel,
        out_shape=jax.ShapeDtypeStruct((WS * WS * WS * WS, NH), _f32),
    )(onehot, rpb_table)
    bias = out.reshape(WS * WS, WS * WS, NH).transpose(2, 0, 1)
    p = jnp.arange(TB)
    r, w = p // W, p % W
    wj = w // WS
    a = r * WS + w % WS
    mask = wj[:, None] == wj[None, :]
    perm = (a[:, None] == jnp.arange(WS * WS)[None, :]).astype(_f32)
    big_p = jnp.einsum('pa,hab,qb->hpq', perm, bias, perm,
                       precision=lax.Precision.HIGHEST)
    big = jnp.where(mask[None], big_p, NEG)
    return big.astype(_f32)


def _attn_kernel(xw_ref, qkvw_ref, qkvb_ref, projw_ref, projb_ref,
                 n1g_ref, n1b_ref, n2g_ref, n2b_ref, gw_ref, gb_ref,
                 bias_ref, xmid_ref, h2_ref, idx_ref):
    x = xw_ref[:, :]
    m = jnp.mean(x, axis=-1, keepdims=True)
    v = jnp.mean((x - m) ** 2, axis=-1, keepdims=True)
    h = (x - m) / jnp.sqrt(v + 1e-5) * n1g_ref[:, :] + n1b_ref[:, :]

    qkv = _bdot(h, qkvw_ref[:, :], ((1,), (1,))) + qkvb_ref[:, :]

    outs = []
    for hd in range(NH):
        q = qkv[:, hd * DH:(hd + 1) * DH] * (DH ** -0.5)
        k = qkv[:, C + hd * DH:C + (hd + 1) * DH]
        vv = qkv[:, 2 * C + hd * DH:2 * C + (hd + 1) * DH]
        s = _bdot(q, k, ((1,), (1,))) + bias_ref[hd]
        mx = jnp.max(s, axis=-1, keepdims=True)
        e = jnp.exp(s - mx)
        p = e / jnp.sum(e, axis=-1, keepdims=True)
        outs.append(_bdot(p, vv, ((1,), (0,))))
    o = jnp.concatenate(outs, axis=1)

    po = _bdot(o, projw_ref[:, :], ((1,), (1,))) + projb_ref[:, :]
    xm = x + po
    xmid_ref[:, :] = xm

    m2 = jnp.mean(xm, axis=-1, keepdims=True)
    v2 = jnp.mean((xm - m2) ** 2, axis=-1, keepdims=True)
    h2 = (xm - m2) / jnp.sqrt(v2 + 1e-5) * n2g_ref[:, :] + n2b_ref[:, :]
    h2_ref[:, :] = h2

    g = _bdot(h2, gw_ref[:, :], ((1,), (1,))) + gb_ref[:, :]
    mxg = jnp.max(g, axis=1, keepdims=True)
    iota = lax.broadcasted_iota(jnp.int32, (TB, E), 1)
    idx = jnp.min(jnp.where(g == mxg, iota, E), axis=1)
    idx_ref[0, 0, :] = idx.astype(jnp.int32)


def _attention(xw, qkv_w, qkv_b, proj_w, proj_b, n1g, n1b, n2g, n2b,
               gate_w, gate_b, big_bias):
    row = lambda a: a.reshape(1, -1)
    wargs = [qkv_w.astype(_bf), row(qkv_b), proj_w.astype(_bf), row(proj_b),
             row(n1g), row(n1b), row(n2g), row(n2b),
             gate_w.astype(_bf), row(gate_b), big_bias]
    full = lambda a: pl.BlockSpec(a.shape, lambda i, nd=a.ndim: (0,) * nd)
    return pl.pallas_call(
        _attn_kernel,
        grid=(N_BLK,),
        in_specs=[pl.BlockSpec((TB, C), lambda i: (i, 0))] +
                 [full(a) for a in wargs],
        out_specs=[
            pl.BlockSpec((TB, C), lambda i: (i, 0)),
            pl.BlockSpec((TB, C), lambda i: (i, 0)),
            pl.BlockSpec((1, 1, TB), lambda i: (i, 0, 0)),
        ],
        out_shape=[
            jax.ShapeDtypeStruct((N_TOK, C), _f32),
            jax.ShapeDtypeStruct((N_TOK, C), _f32),
            jax.ShapeDtypeStruct((N_BLK, 1, TB), jnp.int32),
        ],
    )(xw, *wargs)


def _routing(idx):
    eye = jnp.arange(E, dtype=jnp.int32)
    onehot = (idx[:, None] == eye[None, :]).astype(jnp.int32)
    counts = jnp.sum(onehot, axis=0)
    rank = jnp.cumsum(onehot, axis=0) - onehot
    rank_i = jnp.sum(rank * onehot, axis=1)
    tiles_e = (counts + TM - 1) // TM
    t_end = jnp.cumsum(tiles_e)
    t_off = t_end - tiles_e
    p_off = t_off * TM
    p_off_tok = jnp.sum(onehot * p_off[None, :], axis=1)
    slot = (p_off_tok + rank_i).astype(jnp.int32)

    tt = jnp.arange(MAX_T, dtype=jnp.int32)
    e_t = jnp.minimum(jnp.sum((tt[:, None] >= t_end[None, :]).astype(jnp.int32),
                              axis=1), E - 1).astype(jnp.int32)
    return e_t, slot


_NW = 32


def _sc_gather(table, idx, chunk):
    rows = idx.shape[0]
    d = table.shape[1]
    bpw = rows // _NW
    nst = bpw // chunk
    mesh = plsc.VectorSubcoreMesh(core_axis_name="c", subcore_axis_name="s")

    @functools.partial(
        pl.kernel, mesh=mesh,
        out_type=jax.ShapeDtypeStruct((rows, d), table.dtype),
        scratch_types=[
            pltpu.VMEM((chunk,), jnp.int32),
            pltpu.VMEM((chunk, d), table.dtype),
            pltpu.SemaphoreType.DMA,
        ],
    )
    def k(table_hbm, idx_hbm, out_hbm, idx_v, rows_v, sem):
        wid = lax.axis_index("s") * 2 + lax.axis_index("c")

        def body(j, c):
            base = wid * bpw + j * chunk
            pltpu.sync_copy(idx_hbm.at[pl.ds(base, chunk)], idx_v)
            pltpu.async_copy(table_hbm.at[idx_v], rows_v, sem).wait()
            pltpu.sync_copy(rows_v, out_hbm.at[pl.ds(base, chunk)])
            return c
        lax.fori_loop(0, nst, body, 0)

    return k(table, idx)


def _sc_scatter(rows_hbm, slot, out_rows, chunk):
    n, d = rows_hbm.shape
    bpw = n // _NW
    nst = bpw // chunk
    mesh = plsc.VectorSubcoreMesh(core_axis_name="c", subcore_axis_name="s")

    @functools.partial(
        pl.kernel, mesh=mesh,
        out_type=jax.ShapeDtypeStruct((out_rows, d), rows_hbm.dtype),
        scratch_types=[
            pltpu.VMEM((chunk,), jnp.int32),
            pltpu.VMEM((chunk, d), rows_hbm.dtype),
            pltpu.SemaphoreType.DMA,
        ],
    )
    def k(rows_h, slot_h, out_hbm, idx_v, rows_v, sem):
        wid = lax.axis_index("s") * 2 + lax.axis_index("c")

        def body(j, c):
            base = wid * bpw + j * chunk
            pltpu.sync_copy(slot_h.at[pl.ds(base, chunk)], idx_v)
            pltpu.sync_copy(rows_h.at[pl.ds(base, chunk)], rows_v)
            pltpu.async_copy(rows_v, out_hbm.at[idx_v], sem).wait()
            return c
        lax.fori_loop(0, nst, body, 0)

    return k(rows_hbm, slot)


def _moe_kernel(te_ref, xg_ref, w1_ref, b1_ref, w2_ref, b2_ref, out_ref,
                w1b, w2b):
    t = pl.program_id(0)
    changed = jnp.logical_or(
        t == 0, te_ref[t] != te_ref[jnp.maximum(t - 1, 0)])

    @pl.when(changed)
    def _():
        w1b[:, :] = w1_ref[0].astype(_bf)
        w2b[:, :] = w2_ref[0].astype(_bf)

    x = xg_ref[:, :]
    hmid = _bdot(x, w1b[:, :], ((1,), (1,))) + b1_ref[0]
    a = hmid * 0.5 * (1.0 + lax.erf(hmid * (2.0 ** -0.5)))
    out_ref[:, :] = _dot(a.astype(_bf), w2b[:, :], ((1,), (1,))) + b2_ref[0]


def _moe(xg, fc1_w, fc1_b, fc2_w, fc2_b, te):
    grid_spec = pltpu.PrefetchScalarGridSpec(
        num_scalar_prefetch=1,
        grid=(MAX_T,),
        in_specs=[
            pl.BlockSpec((TM, C), lambda t, te: (t, 0)),
            pl.BlockSpec((1, HID, C), lambda t, te: (te[t], 0, 0)),
            pl.BlockSpec((1, 1, HID), lambda t, te: (te[t], 0, 0)),
            pl.BlockSpec((1, C, HID), lambda t, te: (te[t], 0, 0)),
            pl.BlockSpec((1, 1, C), lambda t, te: (te[t], 0, 0)),
        ],
        out_specs=pl.BlockSpec((TM, C), lambda t, te: (t, 0)),
        scratch_shapes=[
            pltpu.VMEM((HID, C), _bf),
            pltpu.VMEM((C, HID), _bf),
        ],
    )
    return pl.pallas_call(
        _moe_kernel,
        grid_spec=grid_spec,
        out_shape=jax.ShapeDtypeStruct((MAX_T * TM, C), _f32),
    )(te, xg, fc1_w, fc1_b.reshape(E, 1, HID),
      fc2_w, fc2_b.reshape(E, 1, C))


@functools.partial(jax.jit, static_argnums=())
def kernel(x, norm1_g, norm1_b, qkv_w, qkv_b, rpb_table, proj_w, proj_b,
           norm2_g, norm2_b, gate_w, gate_b, fc1_w, fc1_b, fc2_w, fc2_b,
           rel_pos_index):
    big_bias = _rpb_bias(rpb_table, rel_pos_index)
    xw = x.reshape(N_TOK, C)
    xmid, h2, idx3 = _attention(
        xw, qkv_w, qkv_b, proj_w, proj_b, norm1_g, norm1_b,
        norm2_g, norm2_b, gate_w, gate_b, big_bias)
    idx = idx3.reshape(N_TOK)
    te, slot = _routing(idx)
    xg = _sc_scatter(h2, slot, MAX_T * TM, 128)
    moe_sorted = _moe(xg, fc1_w, fc1_b, fc2_w, fc2_b, te)
    moe = _sc_gather(moe_sorted, slot, 128)
    y = (xmid + moe).reshape(B, H * W, C)
    return (y, jnp.float32(0.0))
